# Initial kernel scaffold; baseline (speedup 1.0000x reference)
import jax
import jax.numpy as jnp
from jax.experimental import pallas as pl


def kernel(x, edge_index, edge_weight, W1, b1, W2, b2):
    raise NotImplementedError("write your pallas kernel here")


# speedup vs baseline: 17.6801x; 17.6801x over previous
import functools

import jax
import jax.numpy as jnp
from jax import lax
from jax.experimental import pallas as pl
from jax.experimental.pallas import tpu as pltpu
from jax.experimental.pallas import tpu_sc as plsc

NC = 2
NS = 16
NW = NC * NS
LANES = 16
CH = 128

_mesh = plsc.VectorSubcoreMesh(core_axis_name="c", subcore_axis_name="s")


def _build_deg_kernel(np_, ept):
    nps = np_ // NS
    nchunk = ept // CH

    @functools.partial(
        pl.kernel,
        out_type=jax.ShapeDtypeStruct((NC * np_,), jnp.float32),
        mesh=_mesh,
        compiler_params=pltpu.CompilerParams(needs_layout_passes=False),
        scratch_types=[
            pltpu.VMEM_SHARED((np_,), jnp.float32),
            pltpu.VMEM((ept,), jnp.int32),
            pltpu.VMEM((ept,), jnp.float32),
            pltpu.VMEM((CH,), jnp.int32),
            pltpu.VMEM((CH,), jnp.float32),
            pltpu.VMEM((nps,), jnp.float32),
        ],
    )
    def deg_kernel(dst_hbm, w_hbm, out_hbm, acc, dstv, wv, dstc, wc, zbuf):
        c = lax.axis_index("c")
        s = lax.axis_index("s")
        wid = s * NC + c
        zero = jnp.zeros((LANES,), jnp.float32)

        @pl.loop(0, nps // LANES)
        def _(i):
            zbuf[pl.ds(i * LANES, LANES)] = zero

        pltpu.sync_copy(zbuf, acc.at[pl.ds(s * nps, nps)])
        pltpu.sync_copy(dst_hbm.at[pl.ds(wid * ept, ept)], dstv)
        pltpu.sync_copy(w_hbm.at[pl.ds(wid * ept, ept)], wv)
        plsc.subcore_barrier()

        @pl.loop(0, nchunk)
        def _(k):
            for g in range(CH // LANES):
                dstc[pl.ds(g * LANES, LANES)] = (
                    dstv[pl.ds(k * CH + g * LANES, LANES)])
                wc[pl.ds(g * LANES, LANES)] = (
                    wv[pl.ds(k * CH + g * LANES, LANES)])
            pltpu.sync_copy(wc, acc.at[dstc], add=True)

        plsc.subcore_barrier()
        pltpu.sync_copy(acc.at[pl.ds(s * nps, nps)],
                        out_hbm.at[pl.ds(c * np_ + s * nps, nps)])

    return deg_kernel


def _build_spmm_kernel(np_, d, ept, shift):
    nps = np_ // NS
    nchunk = ept // CH
    mask = (1 << shift) - 1

    @functools.partial(
        pl.kernel,
        out_type=jax.ShapeDtypeStruct((NC, np_, d), jnp.float32),
        mesh=_mesh,
        compiler_params=pltpu.CompilerParams(needs_layout_passes=False),
        scratch_types=[
            pltpu.VMEM_SHARED((np_, d), jnp.float32),
            pltpu.VMEM((np_,), jnp.float32),
            pltpu.VMEM((ept,), jnp.int32),
            pltpu.VMEM((CH,), jnp.int32),
            pltpu.VMEM((CH,), jnp.int32),
            pltpu.VMEM((CH,), jnp.float32),
            pltpu.VMEM((CH,), jnp.float32),
            pltpu.VMEM((CH, d), jnp.float32),
            pltpu.SemaphoreType.DMA,
            pltpu.SemaphoreType.DMA,
        ],
    )
    def spmm_kernel(pk_hbm, w_hbm, dis_hbm, h_hbm, out_hbm,
                    acc, disv, pkv, srcc, dstc, wc, normv, rows, sem, sem2):
        c = lax.axis_index("c")
        s = lax.axis_index("s")
        wid = s * NC + c
        zero = jnp.zeros((LANES,), jnp.float32)

        @pl.loop(0, CH)
        def _(i):
            for j in range(d // LANES):
                rows[i, pl.ds(j * LANES, LANES)] = zero

        @pl.loop(0, nps // CH)
        def _(i):
            pltpu.sync_copy(rows, acc.at[pl.ds(s * nps + i * CH, CH)])

        pltpu.sync_copy(dis_hbm, disv)
        pltpu.sync_copy(pk_hbm.at[pl.ds(wid * ept, ept)], pkv)
        plsc.subcore_barrier()

        @pl.loop(0, nchunk)
        def _(k):
            @pl.loop(0, CH // LANES)
            def _(g):
                p16 = pkv[pl.ds(k * CH + g * LANES, LANES)]
                srcc[pl.ds(g * LANES, LANES)] = p16 & mask

            gat = pltpu.async_copy(h_hbm.at[srcc], rows, sem)
            wcp = pltpu.async_copy(
                w_hbm.at[pl.ds(wid * ept + k * CH, CH)], wc, sem2)

            @pl.loop(0, CH // LANES)
            def _(g):
                p16 = pkv[pl.ds(k * CH + g * LANES, LANES)]
                dstc[pl.ds(g * LANES, LANES)] = (
                    lax.shift_right_logical(p16, shift))

            wcp.wait()

            @pl.loop(0, CH // LANES)
            def _(g):
                s16 = srcc[pl.ds(g * LANES, LANES)]
                d16 = dstc[pl.ds(g * LANES, LANES)]
                w16 = wc[pl.ds(g * LANES, LANES)]
                nv = plsc.load_gather(disv, [s16]) * w16
                normv[pl.ds(g * LANES, LANES)] = (
                    nv * plsc.load_gather(disv, [d16]))

            gat.wait()

            @pl.loop(0, CH)
            def _(i):
                nb = plsc.load_gather(
                    normv, [jnp.zeros((LANES,), jnp.int32) + i])
                for j in range(d // LANES):
                    rows[i, pl.ds(j * LANES, LANES)] = (
                        rows[i, pl.ds(j * LANES, LANES)] * nb)

            pltpu.sync_copy(rows, acc.at[dstc], add=True)

        plsc.subcore_barrier()

        @pl.loop(0, nps // CH)
        def _(i):
            pltpu.sync_copy(acc.at[pl.ds(s * nps + i * CH, CH)],
                            out_hbm.at[c, pl.ds(s * nps + i * CH, CH)])

    return spmm_kernel


def _dis_tc_kernel(deg_ref, o_ref):
    deg = deg_ref[0:1, :] + deg_ref[1:2, :]
    o_ref[...] = jnp.where(deg > 0,
                           lax.rsqrt(jnp.maximum(deg, 1e-12)),
                           0.0)


def _mm_tc_kernel(x_ref, w_ref, o_ref):
    o_ref[...] = lax.dot_general(
        x_ref[...], w_ref[...], (((1,), (1,)), ((), ())),
        preferred_element_type=jnp.float32)


def _final_tc_kernel(s_ref, w2_ref, b1_ref, b2_ref, o_ref):
    a = s_ref[0] + s_ref[1]
    acc = lax.dot_general(a, w2_ref[...], (((1,), (1,)), ((), ())),
                          preferred_element_type=jnp.float32)
    bias = lax.dot_general(b1_ref[...], w2_ref[...], (((1,), (1,)), ((), ())),
                           preferred_element_type=jnp.float32)
    o_ref[...] = acc + bias + b2_ref[...]


def kernel(x, edge_index, edge_weight, W1, b1, W2, b2):
    n, d = x.shape
    e = edge_index.shape[1]

    np_ = ((n + 1023) // 1024) * 1024
    ept = -(-(e + n) // (NW * CH)) * CH
    ep = ept * NW
    pad = ep - e - n

    shift = (n - 1).bit_length()
    src = edge_index[0].astype(jnp.int32)
    dst = edge_index[1].astype(jnp.int32)
    loop_idx = jnp.arange(n, dtype=jnp.int32)
    zpad_i = jnp.zeros((pad,), jnp.int32)
    src_all = jnp.concatenate([src, loop_idx, zpad_i])
    dst_all = jnp.concatenate([dst, loop_idx, zpad_i])
    pk_all = src_all | (dst_all << shift)
    w_all = jnp.concatenate([
        edge_weight.astype(jnp.float32),
        jnp.ones((n,), jnp.float32),
        jnp.zeros((pad,), jnp.float32),
    ])
    x_pad = jnp.pad(x, ((0, np_ - n), (0, 0)))

    deg_parts = _build_deg_kernel(np_, ept)(dst_all, w_all)

    dis = pl.pallas_call(
        _dis_tc_kernel,
        out_shape=jax.ShapeDtypeStruct((1, np_), jnp.float32),
    )(deg_parts.reshape(NC, np_)).reshape(np_)

    bm = 1024
    h = pl.pallas_call(
        _mm_tc_kernel,
        grid=(np_ // bm,),
        in_specs=[pl.BlockSpec((bm, d), lambda i: (i, 0)),
                  pl.BlockSpec((d, d), lambda i: (0, 0))],
        out_specs=pl.BlockSpec((bm, d), lambda i: (i, 0)),
        out_shape=jax.ShapeDtypeStruct((np_, d), jnp.float32),
    )(x_pad, W1)

    s_parts = _build_spmm_kernel(np_, d, ept, shift)(pk_all, w_all, dis, h)

    out_full = pl.pallas_call(
        _final_tc_kernel,
        grid=(np_ // bm,),
        in_specs=[pl.BlockSpec((NC, bm, d), lambda i: (0, i, 0)),
                  pl.BlockSpec((d, d), lambda i: (0, 0)),
                  pl.BlockSpec((1, d), lambda i: (0, 0)),
                  pl.BlockSpec((1, d), lambda i: (0, 0))],
        out_specs=pl.BlockSpec((bm, d), lambda i: (i, 0)),
        out_shape=jax.ShapeDtypeStruct((np_, d), jnp.float32),
    )(s_parts, W2, b1.reshape(1, d), b2.reshape(1, d))

    return out_full[:n]
